# baseline (speedup 1.0000x reference)
import functools

import jax
import jax.numpy as jnp
from jax import lax
from jax.experimental import pallas as pl
from jax.experimental.pallas import tpu as pltpu
from jax.experimental.pallas import tpu_sc as plsc

NBUF = 4
LOOKAHEAD = 2


@functools.lru_cache(maxsize=None)
def _make_lookup(b0: int, b1: int, d_model: int):
    info = plsc.get_sparse_core_info()
    nc, ns = info.num_cores, info.num_subcores
    nw = nc * ns
    n = b0 // nw
    assert n % NBUF == 0
    splits = []
    off = 0
    while off < b1:
        size = min(128, b1 - off)
        splits.append((off, size))
        off += size
    splits = tuple(splits)
    mesh = plsc.VectorSubcoreMesh(core_axis_name="c", subcore_axis_name="s")

    @functools.partial(
        pl.kernel,
        out_type=jax.ShapeDtypeStruct((b0, b1, d_model), jnp.float32),
        mesh=mesh,
        scratch_types=[
            pltpu.VMEM((n, b1), jnp.int32),
            pltpu.VMEM((NBUF, b1, d_model), jnp.float32),
            [pltpu.SemaphoreType.DMA] * NBUF,
            [pltpu.SemaphoreType.DMA] * NBUF,
        ],
        compiler_params=pltpu.CompilerParams(use_tc_tiling_on_sc=False),
    )
    def lookup(idx_hbm, table_hbm, out_hbm, idx_v, rows_v, gsems, ssems):
        wid = lax.axis_index("s") * nc + lax.axis_index("c")
        base = wid * n
        pltpu.sync_copy(idx_hbm.at[pl.ds(base, n)], idx_v)

        def fire_gather(c, b):
            for off, size in splits:
                pltpu.async_copy(
                    table_hbm.at[idx_v.at[c].at[pl.ds(off, size)]],
                    rows_v.at[b].at[pl.ds(off, size)],
                    gsems[b],
                )

        def wait_gather(c, b):
            for off, size in splits:
                pltpu.make_async_copy(
                    table_hbm.at[idx_v.at[c].at[pl.ds(off, size)]],
                    rows_v.at[b].at[pl.ds(off, size)],
                    gsems[b],
                ).wait()

        def fire_store(c, b):
            pltpu.async_copy(rows_v.at[b], out_hbm.at[base + c], ssems[b])

        def wait_store(c, b):
            pltpu.make_async_copy(
                rows_v.at[b], out_hbm.at[base + c], ssems[b]
            ).wait()

        for b in range(NBUF):
            fire_gather(b, b)

        def group(t, carry):
            for b in range(NBUF):
                g = t * NBUF + b
                wait_gather(g, b)
                fire_store(g, b)
                h = g + LOOKAHEAD
                hb = (b + LOOKAHEAD) % NBUF

                @pl.when(jnp.logical_and(h >= NBUF, h < n))
                def _():
                    wait_store(h - NBUF, hb)
                    fire_gather(h, hb)

            return carry

        lax.fori_loop(0, n // NBUF, group, 0)

        for b in range(NBUF):
            c = n - NBUF + b
            wait_store(c, b)

    return lookup


def kernel(x, table):
    b0, b1 = x.shape
    idx = x.astype(jnp.int32)
    return _make_lookup(b0, b1, table.shape[1])(idx, table)


# speedup vs baseline: 1.2205x; 1.2205x over previous
import functools

import jax
import jax.numpy as jnp
from jax import lax
from jax.experimental import pallas as pl
from jax.experimental.pallas import tpu as pltpu
from jax.experimental.pallas import tpu_sc as plsc

NBUF = 4
LOOKAHEAD = 2


@functools.lru_cache(maxsize=None)
def _make_lookup(b0: int, b1: int, d_model: int):
    info = plsc.get_sparse_core_info()
    nc, ns = info.num_cores, info.num_subcores
    nw = nc * ns
    n = b0 // nw
    assert n % NBUF == 0
    splits = []
    off = 0
    while off < b1:
        size = min(128, b1 - off)
        splits.append((off, size))
        off += size
    splits = tuple(splits)
    mesh = plsc.VectorSubcoreMesh(core_axis_name="c", subcore_axis_name="s")

    @functools.partial(
        pl.kernel,
        out_type=jax.ShapeDtypeStruct((b0, b1, 128), jnp.float32),
        mesh=mesh,
        scratch_types=[
            pltpu.VMEM((n, b1), jnp.int32),
            pltpu.VMEM((NBUF, b1, 128), jnp.float32),
            [pltpu.SemaphoreType.DMA] * NBUF,
            [pltpu.SemaphoreType.DMA] * NBUF,
        ],
        compiler_params=pltpu.CompilerParams(use_tc_tiling_on_sc=False),
    )
    def lookup(idx_hbm, table_hbm, out_hbm, idx_v, rows_v, gsems, ssems):
        wid = lax.axis_index("s") * nc + lax.axis_index("c")
        base = wid * n
        pltpu.sync_copy(idx_hbm.at[pl.ds(base, n)], idx_v)

        def fire_gather(c, b):
            for off, size in splits:
                pltpu.async_copy(
                    table_hbm.at[idx_v.at[c].at[pl.ds(off, size)]],
                    rows_v.at[b].at[pl.ds(off, size)],
                    gsems[b],
                )

        def wait_gather(c, b):
            for off, size in splits:
                pltpu.make_async_copy(
                    table_hbm.at[idx_v.at[c].at[pl.ds(off, size)]],
                    rows_v.at[b].at[pl.ds(off, size)],
                    gsems[b],
                ).wait()

        def fire_store(c, b):
            pltpu.async_copy(rows_v.at[b], out_hbm.at[base + c], ssems[b])

        def wait_store(c, b):
            pltpu.make_async_copy(
                rows_v.at[b], out_hbm.at[base + c], ssems[b]
            ).wait()

        for b in range(NBUF):
            fire_gather(b, b)

        def group(t, carry):
            for b in range(NBUF):
                g = t * NBUF + b
                wait_gather(g, b)
                fire_store(g, b)
                h = g + LOOKAHEAD
                hb = (b + LOOKAHEAD) % NBUF

                @pl.when(jnp.logical_and(h >= NBUF, h < n))
                def _():
                    wait_store(h - NBUF, hb)
                    fire_gather(h, hb)

            return carry

        lax.fori_loop(0, n // NBUF, group, 0)

        for b in range(NBUF):
            c = n - NBUF + b
            wait_store(c, b)

    return lookup


def kernel(x, table):
    b0, b1 = x.shape
    idx = x.astype(jnp.int32)
    tpad = jnp.pad(table, ((0, 0), (0, 128 - table.shape[1])))
    out = _make_lookup(b0, b1, table.shape[1])(idx, tpad)
    return out[:, :, : table.shape[1]]


# speedup vs baseline: 1.3102x; 1.0734x over previous
import functools

import jax
import jax.numpy as jnp
from jax import lax
from jax.experimental import pallas as pl
from jax.experimental.pallas import tpu as pltpu
from jax.experimental.pallas import tpu_sc as plsc

NBUF = 4
LOOKAHEAD = 2


@functools.lru_cache(maxsize=None)
def _make_lookup(b0: int, b1: int, d_model: int):
    info = plsc.get_sparse_core_info()
    nc, ns = info.num_cores, info.num_subcores
    nw = nc * ns
    n = b0 // nw
    assert n % NBUF == 0
    splits = []
    off = 0
    while off < b1:
        size = min(128, b1 - off)
        splits.append((off, size))
        off += size
    splits = tuple(splits)
    mesh = plsc.VectorSubcoreMesh(core_axis_name="c", subcore_axis_name="s")

    @functools.partial(
        pl.kernel,
        out_type=jax.ShapeDtypeStruct((b0, b1, 128), jnp.float32),
        mesh=mesh,
        scratch_types=[
            pltpu.VMEM((n, b1), jnp.int32),
            pltpu.VMEM((NBUF, b1, 128), jnp.float32),
            [pltpu.SemaphoreType.DMA] * NBUF,
            [pltpu.SemaphoreType.DMA] * NBUF,
        ],
        compiler_params=pltpu.CompilerParams(use_tc_tiling_on_sc=False),
    )
    def lookup(idx_hbm, table_hbm, out_hbm, idx_v, rows_v, gsems, ssems):
        wid = lax.axis_index("s") * nc + lax.axis_index("c")
        base = wid * n
        pltpu.sync_copy(idx_hbm.at[pl.ds(base, n)], idx_v)

        def fire_gather(c, b):
            for off, size in splits:
                pltpu.async_copy(
                    table_hbm.at[idx_v.at[c].at[pl.ds(off, size)]],
                    rows_v.at[b].at[pl.ds(off, size)],
                    gsems[b],
                )

        def wait_gather(c, b):
            for off, size in splits:
                pltpu.make_async_copy(
                    table_hbm.at[idx_v.at[c].at[pl.ds(off, size)]],
                    rows_v.at[b].at[pl.ds(off, size)],
                    gsems[b],
                ).wait()

        def fire_store(c, b):
            pltpu.async_copy(
                rows_v.at[b].at[:, pl.ds(0, d_model)],
                out_hbm.at[base + c, :, pl.ds(0, d_model)],
                ssems[b],
            )

        def wait_store(c, b):
            pltpu.make_async_copy(
                rows_v.at[b].at[:, pl.ds(0, d_model)],
                out_hbm.at[base + c, :, pl.ds(0, d_model)],
                ssems[b],
            ).wait()

        for b in range(NBUF):
            fire_gather(b, b)

        def group(t, carry):
            for b in range(NBUF):
                g = t * NBUF + b
                wait_gather(g, b)
                fire_store(g, b)
                h = g + LOOKAHEAD
                hb = (b + LOOKAHEAD) % NBUF

                @pl.when(jnp.logical_and(h >= NBUF, h < n))
                def _():
                    wait_store(h - NBUF, hb)
                    fire_gather(h, hb)

            return carry

        lax.fori_loop(0, n // NBUF, group, 0)

        for b in range(NBUF):
            c = n - NBUF + b
            wait_store(c, b)

    return lookup


def kernel(x, table):
    b0, b1 = x.shape
    idx = x.astype(jnp.int32)
    tpad = jnp.pad(table, ((0, 0), (0, 128 - table.shape[1])))
    out = _make_lookup(b0, b1, table.shape[1])(idx, tpad)
    return out[:, :, : table.shape[1]]
